# baseline (speedup 1.0000x reference)
import functools

import jax
import jax.numpy as jnp
from jax import lax
from jax.experimental import pallas as pl
from jax.experimental.pallas import tpu as pltpu
from jax.experimental.pallas import tpu_sc as plsc

POOLN = 7
PTS = POOLN * POOLN
PADC = 104
ROWW = 432
NWORK = 32


def _prep(boxes, positive_indices, shapes):
    (h0, w0), (h1, w1), (h2, w2) = shapes
    B, N = boxes.shape[0], boxes.shape[1]
    nbox = B * N
    fb = boxes.reshape(-1, 4)
    y1, x1, y2, x2 = fb[:, 0], fb[:, 1], fb[:, 2], fb[:, 3]
    h = y2 - y1
    w = x2 - x1
    roi_level = jnp.log(h * w) / jnp.log(2.0)
    lvl = jnp.minimum(5, jnp.maximum(3, jnp.ceil(5.0 + roi_level).astype(jnp.int32)))
    li = lvl - 3

    hm1 = jnp.array([h0 - 1, h1 - 1, h2 - 1], jnp.float32)[li]
    wm1 = jnp.array([w0 - 1, w1 - 1, w2 - 1], jnp.float32)[li]
    p = jnp.arange(POOLN, dtype=jnp.float32)
    in_y = y1[:, None] * hm1[:, None] + p[None, :] * (h * hm1 / (POOLN - 1))[:, None]
    in_x = x1[:, None] * wm1[:, None] + p[None, :] * (w * wm1 / (POOLN - 1))[:, None]
    top = jnp.floor(in_y)
    left = jnp.floor(in_x)
    t = jnp.clip(top, 0, hm1[:, None]).astype(jnp.int32)
    btm = jnp.clip(top + 1.0, 0, hm1[:, None]).astype(jnp.int32)
    lft = jnp.clip(left, 0, wm1[:, None]).astype(jnp.int32)
    rgt = jnp.clip(left + 1.0, 0, wm1[:, None]).astype(jnp.int32)
    yl = in_y - top
    xl = in_x - left
    vy = ((in_y >= 0) & (in_y <= hm1[:, None])).astype(jnp.float32)
    vx = ((in_x >= 0) & (in_x <= wm1[:, None])).astype(jnp.float32)
    pos = (positive_indices.reshape(-1) == 1).astype(jnp.float32)
    m = pos[:, None, None] * (vy[:, :, None] * vx[:, None, :])

    wtl = m * ((1.0 - yl)[:, :, None] * (1.0 - xl)[:, None, :])
    wtr = m * ((1.0 - yl)[:, :, None] * xl[:, None, :])
    wbl = m * (yl[:, :, None] * (1.0 - xl)[:, None, :])
    wbr = m * (yl[:, :, None] * xl[:, None, :])

    Wl = jnp.array([w0, w1, w2], jnp.int32)[li]
    HWl = jnp.array([h0 * w0, h1 * w1, h2 * w2], jnp.int32)[li]
    base = jnp.array([0, B * h0 * w0, B * (h0 * w0 + h1 * w1)], jnp.int32)[li]
    bi = jnp.arange(nbox, dtype=jnp.int32) // N
    base_b = base + bi * HWl
    iy_t = t * Wl[:, None]
    iy_b = btm * Wl[:, None]
    itl = base_b[:, None, None] + iy_t[:, :, None] + lft[:, None, :]
    itr = base_b[:, None, None] + iy_t[:, :, None] + rgt[:, None, :]
    ibl = base_b[:, None, None] + iy_b[:, :, None] + lft[:, None, :]
    ibr = base_b[:, None, None] + iy_b[:, :, None] + rgt[:, None, :]

    def pack(a, b):
        z = jnp.stack([a, b], axis=-1).reshape(nbox, 2 * PTS)
        return jnp.pad(z, ((0, 0), (0, PADC - 2 * PTS)))

    w_all = jnp.stack([wtl, wtr, wbl, wbr], axis=-1).reshape(nbox, 4 * PTS)
    comb = jnp.concatenate(
        [pack(itl, itr), pack(ibl, ibr),
         jax.lax.bitcast_convert_type(w_all, jnp.int32),
         jnp.zeros((nbox, ROWW - 2 * PADC - 4 * PTS), jnp.int32)], axis=1)
    return comb.astype(jnp.int32)


def _splat(vec, c):
    dn = lax.GatherDimensionNumbers(offset_dims=(), collapsed_slice_dims=(0,),
                                    start_index_map=(0,))
    idx = jnp.full((16,), c, jnp.int32)
    return lax.gather(vec, idx[:, None], dn, (1,),
                      mode=lax.GatherScatterMode.PROMISE_IN_BOUNDS)


def _sc_pool(table, comb_all, B, N, C):
    nbox = B * N
    nbox_pad = comb_all.shape[0] // ROWW
    steps = nbox_pad // NWORK - 1
    mesh = plsc.VectorSubcoreMesh(core_axis_name="c", subcore_axis_name="s",
                                  num_cores=2, num_subcores=16)

    @functools.partial(
        pl.kernel,
        out_type=jax.ShapeDtypeStruct((B, N, POOLN, POOLN, C), jnp.float32),
        mesh=mesh,
        scratch_types=[
            pltpu.VMEM((ROWW,), jnp.int32),
            pltpu.VMEM((ROWW,), jnp.int32),
            pltpu.VMEM((PADC, C), jnp.float32),
            pltpu.VMEM((PADC, C), jnp.float32),
            pltpu.VMEM((POOLN, POOLN, C), jnp.float32),
            pltpu.VMEM((POOLN, POOLN, C), jnp.float32),
            pltpu.SemaphoreType.DMA,
            pltpu.SemaphoreType.DMA,
            pltpu.SemaphoreType.DMA,
            pltpu.SemaphoreType.DMA,
            pltpu.SemaphoreType.DMA,
        ],
        compiler_params=pltpu.CompilerParams(needs_layout_passes=False),
    )
    def body(comb_hbm, table_hbm, out_hbm, cv0, cv1, rows_a, rows_b,
             out0, out1, sem_a, sem_b, s_stg, so0, so1):
        wid = lax.axis_index("s") * 2 + lax.axis_index("c")

        pltpu.sync_copy(comb_hbm.at[pl.ds(wid * ROWW, ROWW)], cv0)

        def iter_one(jj, u):
            j = 2 * jj + u
            box = j * NWORK + wid
            cvs = cv0 if u == 0 else cv1
            cvo = cv1 if u == 0 else cv0
            ov = out0 if u == 0 else out1
            so = so0 if u == 0 else so1

            ga = pltpu.async_copy(
                table_hbm.at[cvs.at[pl.ds(0, PADC)]], rows_a, sem_a)
            gb = pltpu.async_copy(
                table_hbm.at[cvs.at[pl.ds(PADC, PADC)]], rows_b, sem_b)
            nxt = (j + 1) * NWORK + wid
            dstg = pltpu.async_copy(comb_hbm.at[pl.ds(nxt * ROWW, ROWW)],
                                    cvo, s_stg)
            ga.wait()
            gb.wait()

            @pl.when(jnp.logical_and(box < nbox, jj >= 1))
            def _():
                pltpu.make_async_copy(ov, out_hbm.at[0, 0], so).wait()

            @pl.when(box < nbox)
            def _():
                def pt_step(p, c2):
                    py = p // POOLN
                    px = p - py * POOLN
                    w16 = plsc.bitcast(cvs[pl.ds(2 * PADC + 4 * p, 16)],
                                       jnp.float32)
                    wtl = _splat(w16, 0)
                    wtr = _splat(w16, 1)
                    wbl = _splat(w16, 2)
                    wbr = _splat(w16, 3)
                    for k in range(C // 16):
                        s = pl.ds(k * 16, 16)
                        acc = (rows_a[2 * p, s] * wtl + rows_a[2 * p + 1, s] * wtr
                               + rows_b[2 * p, s] * wbl + rows_b[2 * p + 1, s] * wbr)
                        ov[py, px, s] = acc
                    return c2

                lax.fori_loop(0, PTS, pt_step, 0)
                bi = box // N
                pltpu.async_copy(ov, out_hbm.at[bi, box - bi * N], so)

            dstg.wait()

        def loop_body(jj, carry):
            iter_one(jj, 0)
            iter_one(jj, 1)
            return carry

        lax.fori_loop(0, steps // 2, loop_body, 0)
        pltpu.make_async_copy(out0, out_hbm.at[0, 0], so0).wait()
        pltpu.make_async_copy(out1, out_hbm.at[0, 0], so1).wait()

    return body(comb_all, table)


def kernel(boxes, positive_indices, feature_maps_0, feature_maps_1,
           feature_maps_2, config):
    B, N = boxes.shape[0], boxes.shape[1]
    C = feature_maps_0.shape[-1]
    nbox = B * N
    shapes = [(f.shape[1], f.shape[2]) for f in
              (feature_maps_0, feature_maps_1, feature_maps_2)]
    comb = _prep(boxes, positive_indices, shapes)
    steps = ((nbox + NWORK - 1) // NWORK + 1) // 2 * 2
    nbox_pad = (steps + 1) * NWORK
    comb = jnp.pad(comb, ((0, nbox_pad - nbox), (0, 0))).reshape(-1)
    table = jnp.concatenate([feature_maps_0.reshape(-1, C),
                             feature_maps_1.reshape(-1, C),
                             feature_maps_2.reshape(-1, C)], axis=0)
    return _sc_pool(table, comb, B, N, C)


# speedup vs baseline: 1.2523x; 1.2523x over previous
import functools

import jax
import jax.numpy as jnp
from jax import lax
from jax.experimental import pallas as pl
from jax.experimental.pallas import tpu as pltpu
from jax.experimental.pallas import tpu_sc as plsc

POOLN = 7
PTS = POOLN * POOLN
PADC = 104
ROWW = 432
NWORK = 32


def _prep(boxes, positive_indices, shapes):
    (h0, w0), (h1, w1), (h2, w2) = shapes
    B, N = boxes.shape[0], boxes.shape[1]
    nbox = B * N
    fb = boxes.reshape(-1, 4)
    y1, x1, y2, x2 = fb[:, 0], fb[:, 1], fb[:, 2], fb[:, 3]
    h = y2 - y1
    w = x2 - x1
    roi_level = jnp.log(h * w) / jnp.log(2.0)
    lvl = jnp.minimum(5, jnp.maximum(3, jnp.ceil(5.0 + roi_level).astype(jnp.int32)))
    li = lvl - 3

    hm1 = jnp.array([h0 - 1, h1 - 1, h2 - 1], jnp.float32)[li]
    wm1 = jnp.array([w0 - 1, w1 - 1, w2 - 1], jnp.float32)[li]
    p = jnp.arange(POOLN, dtype=jnp.float32)
    in_y = y1[:, None] * hm1[:, None] + p[None, :] * (h * hm1 / (POOLN - 1))[:, None]
    in_x = x1[:, None] * wm1[:, None] + p[None, :] * (w * wm1 / (POOLN - 1))[:, None]
    top = jnp.floor(in_y)
    left = jnp.floor(in_x)
    t = jnp.clip(top, 0, hm1[:, None]).astype(jnp.int32)
    btm = jnp.clip(top + 1.0, 0, hm1[:, None]).astype(jnp.int32)
    lft = jnp.clip(left, 0, wm1[:, None]).astype(jnp.int32)
    rgt = jnp.clip(left + 1.0, 0, wm1[:, None]).astype(jnp.int32)
    yl = in_y - top
    xl = in_x - left
    vy = ((in_y >= 0) & (in_y <= hm1[:, None])).astype(jnp.float32)
    vx = ((in_x >= 0) & (in_x <= wm1[:, None])).astype(jnp.float32)
    pos = (positive_indices.reshape(-1) == 1).astype(jnp.float32)
    m = pos[:, None, None] * (vy[:, :, None] * vx[:, None, :])

    wtl = m * ((1.0 - yl)[:, :, None] * (1.0 - xl)[:, None, :])
    wtr = m * ((1.0 - yl)[:, :, None] * xl[:, None, :])
    wbl = m * (yl[:, :, None] * (1.0 - xl)[:, None, :])
    wbr = m * (yl[:, :, None] * xl[:, None, :])

    Wl = jnp.array([w0, w1, w2], jnp.int32)[li]
    HWl = jnp.array([h0 * w0, h1 * w1, h2 * w2], jnp.int32)[li]
    base = jnp.array([0, B * h0 * w0, B * (h0 * w0 + h1 * w1)], jnp.int32)[li]
    bi = jnp.arange(nbox, dtype=jnp.int32) // N
    base_b = base + bi * HWl
    iy_t = t * Wl[:, None]
    iy_b = btm * Wl[:, None]
    itl = base_b[:, None, None] + iy_t[:, :, None] + lft[:, None, :]
    itr = base_b[:, None, None] + iy_t[:, :, None] + rgt[:, None, :]
    ibl = base_b[:, None, None] + iy_b[:, :, None] + lft[:, None, :]
    ibr = base_b[:, None, None] + iy_b[:, :, None] + rgt[:, None, :]

    def pack(a, b):
        z = jnp.stack([a, b], axis=-1).reshape(nbox, 2 * PTS)
        return jnp.pad(z, ((0, 0), (0, PADC - 2 * PTS)))

    w_all = jnp.stack([wtl, wtr, wbl, wbr], axis=-1).reshape(nbox, 4 * PTS)
    comb = jnp.concatenate(
        [pack(itl, itr), pack(ibl, ibr),
         jax.lax.bitcast_convert_type(w_all, jnp.int32),
         jnp.zeros((nbox, ROWW - 2 * PADC - 4 * PTS), jnp.int32)], axis=1)
    return comb.astype(jnp.int32)


def _splat(vec, c):
    dn = lax.GatherDimensionNumbers(offset_dims=(), collapsed_slice_dims=(0,),
                                    start_index_map=(0,))
    idx = jnp.full((16,), c, jnp.int32)
    return lax.gather(vec, idx[:, None], dn, (1,),
                      mode=lax.GatherScatterMode.PROMISE_IN_BOUNDS)


def _sc_pool(table, comb_all, B, N, C):
    nbox = B * N
    nbox_pad = comb_all.shape[0] // ROWW
    steps = nbox_pad // NWORK
    mesh = plsc.VectorSubcoreMesh(core_axis_name="c", subcore_axis_name="s",
                                  num_cores=2, num_subcores=16)

    @functools.partial(
        pl.kernel,
        out_type=jax.ShapeDtypeStruct((B, N, POOLN, POOLN, C), jnp.float32),
        mesh=mesh,
        scratch_types=[
            pltpu.VMEM((ROWW,), jnp.int32),
            pltpu.VMEM((PADC, C), jnp.float32),
            pltpu.VMEM((PADC, C), jnp.float32),
            pltpu.VMEM((POOLN, POOLN, C), jnp.float32),
            pltpu.SemaphoreType.DMA,
            pltpu.SemaphoreType.DMA,
        ],
        compiler_params=pltpu.CompilerParams(needs_layout_passes=False),
    )
    def body(comb_hbm, table_hbm, out_hbm, cv, rows_a, rows_b, out_v,
             sem_a, sem_b):
        wid = lax.axis_index("s") * 2 + lax.axis_index("c")

        def box_step(j, carry):
            box = j * NWORK + wid

            @pl.when(box < nbox)
            def _():
                pltpu.sync_copy(comb_hbm.at[pl.ds(box * ROWW, ROWW)], cv)
                cp_a = pltpu.async_copy(
                    table_hbm.at[cv.at[pl.ds(0, PADC)]], rows_a, sem_a)
                cp_b = pltpu.async_copy(
                    table_hbm.at[cv.at[pl.ds(PADC, PADC)]], rows_b, sem_b)
                cp_a.wait()
                cp_b.wait()

                def pt_step(p, c2):
                    py = p // POOLN
                    px = p - py * POOLN
                    w16 = plsc.bitcast(cv[pl.ds(2 * PADC + 4 * p, 16)],
                                       jnp.float32)
                    wtl = _splat(w16, 0)
                    wtr = _splat(w16, 1)
                    wbl = _splat(w16, 2)
                    wbr = _splat(w16, 3)
                    for k in range(C // 16):
                        s = pl.ds(k * 16, 16)
                        acc = (rows_a[2 * p, s] * wtl + rows_a[2 * p + 1, s] * wtr
                               + rows_b[2 * p, s] * wbl + rows_b[2 * p + 1, s] * wbr)
                        out_v[py, px, s] = acc
                    return c2

                lax.fori_loop(0, PTS, pt_step, 0)
                bi = box // N
                pltpu.sync_copy(out_v, out_hbm.at[bi, box - bi * N])

            return carry

        lax.fori_loop(0, steps, box_step, 0)

    return body(comb_all, table)


def kernel(boxes, positive_indices, feature_maps_0, feature_maps_1,
           feature_maps_2, config):
    B, N = boxes.shape[0], boxes.shape[1]
    C = feature_maps_0.shape[-1]
    nbox = B * N
    shapes = [(f.shape[1], f.shape[2]) for f in
              (feature_maps_0, feature_maps_1, feature_maps_2)]
    comb = _prep(boxes, positive_indices, shapes)
    nbox_pad = ((nbox + NWORK - 1) // NWORK) * NWORK
    comb = jnp.pad(comb, ((0, nbox_pad - nbox), (0, 0))).reshape(-1)
    table = jnp.concatenate([feature_maps_0.reshape(-1, C),
                             feature_maps_1.reshape(-1, C),
                             feature_maps_2.reshape(-1, C)], axis=0)
    return _sc_pool(table, comb, B, N, C)
